# Initial kernel scaffold; baseline (speedup 1.0000x reference)
import jax
import jax.numpy as jnp
from jax.experimental import pallas as pl


def kernel(x, edge_index, W1, b1, W2, b2, W3, b3):
    raise NotImplementedError("write your pallas kernel here")


# speedup vs baseline: 16.2701x; 16.2701x over previous
import functools

import jax
import jax.numpy as jnp
from jax import lax
from jax.experimental import pallas as pl
from jax.experimental.pallas import tpu as pltpu
from jax.experimental.pallas import tpu_sc as plsc

N = 10000
D = 128
E = 320000

NC = 2
NS = 16
NW = NC * NS
EPT = E // NW
K = 80
C = EPT // K
IO_T = 10
RPT = N // IO_T
ZR = 40

_mesh = plsc.VectorSubcoreMesh(core_axis_name="c", subcore_axis_name="s")


@functools.partial(
    pl.kernel,
    out_type=jax.ShapeDtypeStruct((NC, N, D), jnp.float32),
    mesh=_mesh,
    scratch_types=[
        pltpu.VMEM((C, K), jnp.int32),
        pltpu.VMEM((C, K), jnp.int32),
        pltpu.VMEM((K, D), jnp.float32),
        pltpu.VMEM((ZR, D), jnp.float32),
        pltpu.VMEM_SHARED((N, D), jnp.float32),
        pltpu.SemaphoreType.DMA,
    ],
)
def _sc_aggregate(h_hbm, src_hbm, dst_hbm, zeros_hbm, out_hbm,
                  sidx, didx, rows, zbuf, acc, sem):
    cid = lax.axis_index("c")
    sid = lax.axis_index("s")
    w = cid * NS + sid

    @pl.when(sid < IO_T)
    def _():
        pltpu.sync_copy(zeros_hbm, zbuf)
        for j in range(RPT // ZR):
            pltpu.sync_copy(zbuf, acc.at[pl.ds(sid * RPT + j * ZR, ZR)])

    pltpu.sync_copy(src_hbm.at[w], sidx)
    pltpu.sync_copy(dst_hbm.at[w], didx)
    plsc.subcore_barrier()

    def chunk(c, carry):
        pltpu.async_copy(h_hbm.at[sidx.at[c]], rows, sem).wait()
        pltpu.sync_copy(rows, acc.at[didx.at[c]], add=True)
        return carry

    lax.fori_loop(0, C, chunk, 0)
    plsc.subcore_barrier()

    @pl.when(sid < IO_T)
    def _():
        for j in range(RPT // ZR):
            pltpu.sync_copy(acc.at[pl.ds(sid * RPT + j * ZR, ZR)], zbuf)
            pltpu.sync_copy(zbuf, out_hbm.at[cid, pl.ds(sid * RPT + j * ZR, ZR)])


@functools.partial(
    pl.kernel,
    out_type=jax.ShapeDtypeStruct((NC * N,), jnp.float32),
    mesh=_mesh,
    scratch_types=[
        pltpu.VMEM((C, K), jnp.int32),
        pltpu.VMEM((K,), jnp.float32),
        pltpu.VMEM((RPT,), jnp.float32),
        pltpu.VMEM_SHARED((N,), jnp.float32),
    ],
)
def _sc_degree(dst_hbm, zeros_hbm, out_hbm, didx, ones_v, dbuf, acc):
    cid = lax.axis_index("c")
    sid = lax.axis_index("s")
    w = cid * NS + sid

    @pl.when(sid < IO_T)
    def _():
        pltpu.sync_copy(zeros_hbm, dbuf)
        pltpu.sync_copy(dbuf, acc.at[pl.ds(sid * RPT, RPT)])

    pltpu.sync_copy(dst_hbm.at[w], didx)
    for i in range(K // 16):
        ones_v[pl.ds(i * 16, 16)] = jnp.ones((16,), jnp.float32)
    plsc.subcore_barrier()

    def chunk(c, carry):
        pltpu.sync_copy(ones_v, acc.at[didx.at[c]], add=True)
        return carry

    lax.fori_loop(0, C, chunk, 0)
    plsc.subcore_barrier()

    @pl.when(sid < IO_T)
    def _():
        pltpu.sync_copy(acc.at[pl.ds(sid * RPT, RPT)], dbuf)
        pltpu.sync_copy(dbuf, out_hbm.at[pl.ds(cid * N + sid * RPT, RPT)])


_BM = 1000


def _dinv_block(dref):
    d = dref[...]
    deg = d[:, 0:1] + d[:, 1:2] + 1.0
    return lax.rsqrt(jnp.maximum(deg, 1.0))


def _prep_body(xref, wref, dref, out):
    out[...] = jnp.dot(xref[...], wref[...],
                       preferred_element_type=jnp.float32) * _dinv_block(dref)


def _layer_body(aref, href, wref, bref, dref, out):
    dinv = _dinv_block(dref)
    p = aref[0] + aref[1] + href[...]
    t = jnp.maximum(p * dinv + bref[...], 0.0)
    out[...] = jnp.dot(t, wref[...],
                       preferred_element_type=jnp.float32) * dinv


def _final_body(aref, href, bref, dref, out):
    dinv = _dinv_block(dref)
    out[...] = (aref[0] + aref[1] + href[...]) * dinv + bref[...]


_row_spec = pl.BlockSpec((_BM, D), lambda i: (i, 0))
_agg_spec = pl.BlockSpec((NC, _BM, D), lambda i: (0, i, 0))
_w_spec = pl.BlockSpec((D, D), lambda i: (0, 0))
_b_spec = pl.BlockSpec((1, D), lambda i: (0, 0))
_deg_spec = pl.BlockSpec((_BM, NC), lambda i: (i, 0))
_out_f32 = jax.ShapeDtypeStruct((N, D), jnp.float32)

_prep = pl.pallas_call(
    _prep_body, grid=(N // _BM,),
    in_specs=[_row_spec, _w_spec, _deg_spec],
    out_specs=_row_spec, out_shape=_out_f32)

_layer = pl.pallas_call(
    _layer_body, grid=(N // _BM,),
    in_specs=[_agg_spec, _row_spec, _w_spec, _b_spec, _deg_spec],
    out_specs=_row_spec, out_shape=_out_f32)

_final = pl.pallas_call(
    _final_body, grid=(N // _BM,),
    in_specs=[_agg_spec, _row_spec, _b_spec, _deg_spec],
    out_specs=_row_spec, out_shape=_out_f32)


def kernel(x, edge_index, W1, b1, W2, b2, W3, b3):
    src = edge_index[0].reshape(NW, C, K)
    dst = edge_index[1].reshape(NW, C, K)
    zeros_rows = jnp.zeros((ZR, D), jnp.float32)
    zeros_deg = jnp.zeros((RPT,), jnp.float32)
    b1r = b1.reshape(1, D)
    b2r = b2.reshape(1, D)
    b3r = b3.reshape(1, D)

    degp = _sc_degree(dst, zeros_deg).reshape(NC, N).T
    h1 = _prep(x, W1, degp)
    a1 = _sc_aggregate(h1, src, dst, zeros_rows)
    h2 = _layer(a1, h1, W2, b1r, degp)
    a2 = _sc_aggregate(h2, src, dst, zeros_rows)
    h3 = _layer(a2, h2, W3, b2r, degp)
    a3 = _sc_aggregate(h3, src, dst, zeros_rows)
    return _final(a3, h3, b3r, degp)


# speedup vs baseline: 20.4124x; 1.2546x over previous
import functools

import jax
import jax.numpy as jnp
from jax import lax
from jax.experimental import pallas as pl
from jax.experimental.pallas import tpu as pltpu
from jax.experimental.pallas import tpu_sc as plsc

N = 10000
D = 128
E = 320000

NC = 2
NS = 16
NW = NC * NS
EPT = E // NW
K = 80
C = EPT // K
SEG = 5
CS = C // SEG
IO_T = 10
RPT = N // IO_T
ZR = 40

_mesh = plsc.VectorSubcoreMesh(core_axis_name="c", subcore_axis_name="s")


@functools.partial(
    pl.kernel,
    out_type=jax.ShapeDtypeStruct((NC, N, D), jnp.float32),
    mesh=_mesh,
    scratch_types=[
        pltpu.VMEM((CS, K), jnp.int32),
        pltpu.VMEM((CS, K), jnp.int32),
        pltpu.VMEM((K, D), jnp.float32),
        pltpu.VMEM((K, D), jnp.float32),
        pltpu.VMEM_SHARED((N, D), jnp.float32),
        pltpu.SemaphoreType.DMA,
        pltpu.SemaphoreType.DMA,
        pltpu.SemaphoreType.DMA,
        pltpu.SemaphoreType.DMA,
    ],
)
def _sc_aggregate(h_hbm, src_hbm, dst_hbm, zeros_hbm, out_hbm,
                  sidx, didx, rows0, rows1, acc, gsem0, gsem1, ssem0, ssem1):
    cid = lax.axis_index("c")
    sid = lax.axis_index("s")
    w = cid * NS + sid

    @pl.when(sid < IO_T)
    def _():
        z = rows0.at[pl.ds(0, ZR)]
        pltpu.sync_copy(zeros_hbm, z)
        for j in range(RPT // ZR):
            pltpu.async_copy(z, acc.at[pl.ds(sid * RPT + j * ZR, ZR)], ssem0)
        for j in range(RPT // ZR):
            pltpu.make_async_copy(
                z, acc.at[pl.ds(sid * RPT + j * ZR, ZR)], ssem0).wait()

    plsc.subcore_barrier()

    def pair(i, carry):
        c0 = 2 * i
        pltpu.make_async_copy(h_hbm.at[sidx.at[c0]], rows0, gsem0).wait()
        pltpu.async_copy(h_hbm.at[sidx.at[c0 + 1]], rows1, gsem1)
        pltpu.async_copy(rows0, acc.at[didx.at[c0]], ssem0, add=True)
        pltpu.make_async_copy(h_hbm.at[sidx.at[c0 + 1]], rows1, gsem1).wait()
        pltpu.make_async_copy(rows0, acc.at[didx.at[c0]], ssem0).wait()
        pltpu.async_copy(h_hbm.at[sidx.at[c0 + 2]], rows0, gsem0)
        pltpu.async_copy(rows1, acc.at[didx.at[c0 + 1]], ssem1, add=True)
        pltpu.make_async_copy(rows1, acc.at[didx.at[c0 + 1]], ssem1).wait()
        return carry

    for s in range(SEG):
        pltpu.sync_copy(src_hbm.at[w, s], sidx)
        pltpu.sync_copy(dst_hbm.at[w, s], didx)
        pltpu.async_copy(h_hbm.at[sidx.at[0]], rows0, gsem0)
        lax.fori_loop(0, (CS - 1) // 2, pair, 0)
        pltpu.make_async_copy(h_hbm.at[sidx.at[CS - 1]], rows0, gsem0).wait()
        pltpu.async_copy(rows0, acc.at[didx.at[CS - 1]], ssem0, add=True)
        pltpu.make_async_copy(rows0, acc.at[didx.at[CS - 1]], ssem0).wait()
    plsc.subcore_barrier()

    @pl.when(sid < IO_T)
    def _():
        bufs = (rows0.at[pl.ds(0, ZR)], rows1.at[pl.ds(0, ZR)])
        sems = (ssem0, ssem1)
        nj = RPT // ZR
        for j in range(nj):
            b, s = bufs[j % 2], sems[j % 2]
            if j >= 2:
                pltpu.make_async_copy(
                    b, out_hbm.at[cid, pl.ds(sid * RPT + (j - 2) * ZR, ZR)],
                    s).wait()
            pltpu.sync_copy(acc.at[pl.ds(sid * RPT + j * ZR, ZR)], b)
            pltpu.async_copy(
                b, out_hbm.at[cid, pl.ds(sid * RPT + j * ZR, ZR)], s)
        for j in (nj - 2, nj - 1):
            pltpu.make_async_copy(
                bufs[j % 2], out_hbm.at[cid, pl.ds(sid * RPT + j * ZR, ZR)],
                sems[j % 2]).wait()


@functools.partial(
    pl.kernel,
    out_type=jax.ShapeDtypeStruct((NC * N,), jnp.float32),
    mesh=_mesh,
    scratch_types=[
        pltpu.VMEM((C, K), jnp.int32),
        pltpu.VMEM((K,), jnp.float32),
        pltpu.VMEM((RPT,), jnp.float32),
        pltpu.VMEM_SHARED((N,), jnp.float32),
    ],
)
def _sc_degree(dst_hbm, zeros_hbm, out_hbm, didx, ones_v, dbuf, acc):
    cid = lax.axis_index("c")
    sid = lax.axis_index("s")
    w = cid * NS + sid

    @pl.when(sid < IO_T)
    def _():
        pltpu.sync_copy(zeros_hbm, dbuf)
        pltpu.sync_copy(dbuf, acc.at[pl.ds(sid * RPT, RPT)])

    pltpu.sync_copy(dst_hbm.at[w], didx)
    for i in range(K // 16):
        ones_v[pl.ds(i * 16, 16)] = jnp.ones((16,), jnp.float32)
    plsc.subcore_barrier()

    def chunk(c, carry):
        pltpu.sync_copy(ones_v, acc.at[didx.at[c]], add=True)
        return carry

    lax.fori_loop(0, C, chunk, 0)
    plsc.subcore_barrier()

    @pl.when(sid < IO_T)
    def _():
        pltpu.sync_copy(acc.at[pl.ds(sid * RPT, RPT)], dbuf)
        pltpu.sync_copy(dbuf, out_hbm.at[pl.ds(cid * N + sid * RPT, RPT)])


_BM = 1000


def _dinv_block(dref):
    d = dref[...]
    deg = d[:, 0:1] + d[:, 1:2] + 1.0
    return lax.rsqrt(jnp.maximum(deg, 1.0))


def _prep_body(xref, wref, dref, out):
    out[...] = jnp.dot(xref[...], wref[...],
                       preferred_element_type=jnp.float32) * _dinv_block(dref)


def _layer_body(aref, href, wref, bref, dref, out):
    dinv = _dinv_block(dref)
    p = aref[0] + aref[1] + href[...]
    t = jnp.maximum(p * dinv + bref[...], 0.0)
    out[...] = jnp.dot(t, wref[...],
                       preferred_element_type=jnp.float32) * dinv


def _final_body(aref, href, bref, dref, out):
    dinv = _dinv_block(dref)
    out[...] = (aref[0] + aref[1] + href[...]) * dinv + bref[...]


_row_spec = pl.BlockSpec((_BM, D), lambda i: (i, 0))
_agg_spec = pl.BlockSpec((NC, _BM, D), lambda i: (0, i, 0))
_w_spec = pl.BlockSpec((D, D), lambda i: (0, 0))
_b_spec = pl.BlockSpec((1, D), lambda i: (0, 0))
_deg_spec = pl.BlockSpec((_BM, NC), lambda i: (i, 0))
_out_f32 = jax.ShapeDtypeStruct((N, D), jnp.float32)

_prep = pl.pallas_call(
    _prep_body, grid=(N // _BM,),
    in_specs=[_row_spec, _w_spec, _deg_spec],
    out_specs=_row_spec, out_shape=_out_f32)

_layer = pl.pallas_call(
    _layer_body, grid=(N // _BM,),
    in_specs=[_agg_spec, _row_spec, _w_spec, _b_spec, _deg_spec],
    out_specs=_row_spec, out_shape=_out_f32)

_final = pl.pallas_call(
    _final_body, grid=(N // _BM,),
    in_specs=[_agg_spec, _row_spec, _b_spec, _deg_spec],
    out_specs=_row_spec, out_shape=_out_f32)


def kernel(x, edge_index, W1, b1, W2, b2, W3, b3):
    src = edge_index[0].reshape(NW, SEG, CS, K)
    dst = edge_index[1].reshape(NW, SEG, CS, K)
    dst_flat = edge_index[1].reshape(NW, C, K)
    zeros_rows = jnp.zeros((ZR, D), jnp.float32)
    zeros_deg = jnp.zeros((RPT,), jnp.float32)
    b1r = b1.reshape(1, D)
    b2r = b2.reshape(1, D)
    b3r = b3.reshape(1, D)

    degp = _sc_degree(dst_flat, zeros_deg).reshape(NC, N).T
    h1 = _prep(x, W1, degp)
    a1 = _sc_aggregate(h1, src, dst, zeros_rows)
    h2 = _layer(a1, h1, W2, b1r, degp)
    a2 = _sc_aggregate(h2, src, dst, zeros_rows)
    h3 = _layer(a2, h2, W3, b2r, degp)
    a3 = _sc_aggregate(h3, src, dst, zeros_rows)
    return _final(a3, h3, b3r, degp)


# speedup vs baseline: 24.2419x; 1.1876x over previous
import functools

import jax
import jax.numpy as jnp
from jax import lax
from jax.experimental import pallas as pl
from jax.experimental.pallas import tpu as pltpu
from jax.experimental.pallas import tpu_sc as plsc

N = 10000
D = 128
E = 320000

NC = 2
NS = 16
NW = NC * NS
EPT = E // NW
K = 80
C = EPT // K
SEG = 5
CS = C // SEG
IO_T = 10
RPT = N // IO_T
ZR = 40

_mesh = plsc.VectorSubcoreMesh(core_axis_name="c", subcore_axis_name="s")


@functools.partial(
    pl.kernel,
    out_type=jax.ShapeDtypeStruct((NC, N, D), jnp.float32),
    mesh=_mesh,
    scratch_types=[
        pltpu.VMEM((CS, K), jnp.int32),
        pltpu.VMEM((CS, K), jnp.int32),
        pltpu.VMEM((K, D), jnp.float32),
        pltpu.VMEM((K, D), jnp.float32),
        pltpu.VMEM_SHARED((N, D), jnp.float32),
        pltpu.SemaphoreType.DMA,
        pltpu.SemaphoreType.DMA,
        pltpu.SemaphoreType.DMA,
        pltpu.SemaphoreType.DMA,
    ],
)
def _sc_aggregate(h_hbm, src_hbm, dst_hbm, zeros_hbm, out_hbm,
                  sidx, didx, rows0, rows1, acc, gsem0, gsem1, ssem0, ssem1):
    cid = lax.axis_index("c")
    sid = lax.axis_index("s")
    w = cid * NS + sid

    @pl.when(sid < IO_T)
    def _():
        z = rows0.at[pl.ds(0, ZR)]
        pltpu.sync_copy(zeros_hbm, z)
        for j in range(RPT // ZR):
            pltpu.async_copy(z, acc.at[pl.ds(sid * RPT + j * ZR, ZR)], ssem0)
        for j in range(RPT // ZR):
            pltpu.make_async_copy(
                z, acc.at[pl.ds(sid * RPT + j * ZR, ZR)], ssem0).wait()

    plsc.subcore_barrier()

    def pair(i, carry):
        c0 = 2 * i
        pltpu.make_async_copy(h_hbm.at[sidx.at[c0]], rows0, gsem0).wait()
        pltpu.async_copy(rows0, acc.at[didx.at[c0]], ssem0, add=True)
        pltpu.make_async_copy(h_hbm.at[sidx.at[c0 + 1]], rows1, gsem1).wait()
        pltpu.make_async_copy(rows0, acc.at[didx.at[c0]], ssem0).wait()
        pltpu.async_copy(h_hbm.at[sidx.at[c0 + 2]], rows0, gsem0)
        pltpu.async_copy(rows1, acc.at[didx.at[c0 + 1]], ssem1, add=True)
        pltpu.make_async_copy(rows1, acc.at[didx.at[c0 + 1]], ssem1).wait()
        c3 = jnp.minimum(c0 + 3, CS - 1)
        pltpu.async_copy(h_hbm.at[sidx.at[c3]], rows1, gsem1)
        return carry

    for s in range(SEG):
        pltpu.sync_copy(src_hbm.at[w, s], sidx)
        pltpu.sync_copy(dst_hbm.at[w, s], didx)
        pltpu.async_copy(h_hbm.at[sidx.at[0]], rows0, gsem0)
        pltpu.async_copy(h_hbm.at[sidx.at[1]], rows1, gsem1)
        lax.fori_loop(0, (CS - 1) // 2, pair, 0)
        pltpu.make_async_copy(h_hbm.at[sidx.at[CS - 1]], rows0, gsem0).wait()
        pltpu.async_copy(rows0, acc.at[didx.at[CS - 1]], ssem0, add=True)
        pltpu.make_async_copy(h_hbm.at[sidx.at[CS - 1]], rows1, gsem1).wait()
        pltpu.make_async_copy(rows0, acc.at[didx.at[CS - 1]], ssem0).wait()
    plsc.subcore_barrier()

    @pl.when(sid < IO_T)
    def _():
        bufs = (rows0.at[pl.ds(0, ZR)], rows1.at[pl.ds(0, ZR)])
        sems = (ssem0, ssem1)
        nj = RPT // ZR
        for j in range(nj):
            b, s = bufs[j % 2], sems[j % 2]
            if j >= 2:
                pltpu.make_async_copy(
                    b, out_hbm.at[cid, pl.ds(sid * RPT + (j - 2) * ZR, ZR)],
                    s).wait()
            pltpu.sync_copy(acc.at[pl.ds(sid * RPT + j * ZR, ZR)], b)
            pltpu.async_copy(
                b, out_hbm.at[cid, pl.ds(sid * RPT + j * ZR, ZR)], s)
        for j in (nj - 2, nj - 1):
            pltpu.make_async_copy(
                bufs[j % 2], out_hbm.at[cid, pl.ds(sid * RPT + j * ZR, ZR)],
                sems[j % 2]).wait()


@functools.partial(
    pl.kernel,
    out_type=jax.ShapeDtypeStruct((NC * N,), jnp.float32),
    mesh=_mesh,
    scratch_types=[
        pltpu.VMEM((C, K), jnp.int32),
        pltpu.VMEM((K,), jnp.float32),
        pltpu.VMEM((RPT,), jnp.float32),
        pltpu.VMEM_SHARED((N,), jnp.float32),
    ],
)
def _sc_degree(dst_hbm, zeros_hbm, out_hbm, didx, ones_v, dbuf, acc):
    cid = lax.axis_index("c")
    sid = lax.axis_index("s")
    w = cid * NS + sid

    @pl.when(sid < IO_T)
    def _():
        pltpu.sync_copy(zeros_hbm, dbuf)
        pltpu.sync_copy(dbuf, acc.at[pl.ds(sid * RPT, RPT)])

    pltpu.sync_copy(dst_hbm.at[w], didx)
    for i in range(K // 16):
        ones_v[pl.ds(i * 16, 16)] = jnp.ones((16,), jnp.float32)
    plsc.subcore_barrier()

    def chunk(c, carry):
        pltpu.sync_copy(ones_v, acc.at[didx.at[c]], add=True)
        return carry

    lax.fori_loop(0, C, chunk, 0)
    plsc.subcore_barrier()

    @pl.when(sid < IO_T)
    def _():
        pltpu.sync_copy(acc.at[pl.ds(sid * RPT, RPT)], dbuf)
        pltpu.sync_copy(dbuf, out_hbm.at[pl.ds(cid * N + sid * RPT, RPT)])


_BM = 1000


def _dinv_block(dref):
    d = dref[...]
    deg = d[:, 0:1] + d[:, 1:2] + 1.0
    return lax.rsqrt(jnp.maximum(deg, 1.0))


def _prep_body(xref, wref, dref, out):
    out[...] = jnp.dot(xref[...], wref[...],
                       preferred_element_type=jnp.float32) * _dinv_block(dref)


def _layer_body(aref, href, wref, bref, dref, out):
    dinv = _dinv_block(dref)
    p = aref[0] + aref[1] + href[...]
    t = jnp.maximum(p * dinv + bref[...], 0.0)
    out[...] = jnp.dot(t, wref[...],
                       preferred_element_type=jnp.float32) * dinv


def _final_body(aref, href, bref, dref, out):
    dinv = _dinv_block(dref)
    out[...] = (aref[0] + aref[1] + href[...]) * dinv + bref[...]


_row_spec = pl.BlockSpec((_BM, D), lambda i: (i, 0))
_agg_spec = pl.BlockSpec((NC, _BM, D), lambda i: (0, i, 0))
_w_spec = pl.BlockSpec((D, D), lambda i: (0, 0))
_b_spec = pl.BlockSpec((1, D), lambda i: (0, 0))
_deg_spec = pl.BlockSpec((_BM, NC), lambda i: (i, 0))
_out_f32 = jax.ShapeDtypeStruct((N, D), jnp.float32)

_prep = pl.pallas_call(
    _prep_body, grid=(N // _BM,),
    in_specs=[_row_spec, _w_spec, _deg_spec],
    out_specs=_row_spec, out_shape=_out_f32)

_layer = pl.pallas_call(
    _layer_body, grid=(N // _BM,),
    in_specs=[_agg_spec, _row_spec, _w_spec, _b_spec, _deg_spec],
    out_specs=_row_spec, out_shape=_out_f32)

_final = pl.pallas_call(
    _final_body, grid=(N // _BM,),
    in_specs=[_agg_spec, _row_spec, _b_spec, _deg_spec],
    out_specs=_row_spec, out_shape=_out_f32)


def kernel(x, edge_index, W1, b1, W2, b2, W3, b3):
    src = edge_index[0].reshape(NW, SEG, CS, K)
    dst = edge_index[1].reshape(NW, SEG, CS, K)
    dst_flat = edge_index[1].reshape(NW, C, K)
    zeros_rows = jnp.zeros((ZR, D), jnp.float32)
    zeros_deg = jnp.zeros((RPT,), jnp.float32)
    b1r = b1.reshape(1, D)
    b2r = b2.reshape(1, D)
    b3r = b3.reshape(1, D)

    degp = _sc_degree(dst_flat, zeros_deg).reshape(NC, N).T
    h1 = _prep(x, W1, degp)
    a1 = _sc_aggregate(h1, src, dst, zeros_rows)
    h2 = _layer(a1, h1, W2, b1r, degp)
    a2 = _sc_aggregate(h2, src, dst, zeros_rows)
    h3 = _layer(a2, h2, W3, b2r, degp)
    a3 = _sc_aggregate(h3, src, dst, zeros_rows)
    return _final(a3, h3, b3r, degp)


# speedup vs baseline: 24.5288x; 1.0118x over previous
import functools

import jax
import jax.numpy as jnp
from jax import lax
from jax.experimental import pallas as pl
from jax.experimental.pallas import tpu as pltpu
from jax.experimental.pallas import tpu_sc as plsc

N = 10000
D = 128
E = 320000

NC = 2
NS = 16
NW = NC * NS
EPT = E // NW
K = 40
C = EPT // K
SEG = 5
CS = C // SEG
NB = 4
KD = 80
CD = EPT // KD
IO_T = 10
RPT = N // IO_T
ZR = 40

_mesh = plsc.VectorSubcoreMesh(core_axis_name="c", subcore_axis_name="s")


@functools.partial(
    pl.kernel,
    out_type=jax.ShapeDtypeStruct((NC, N, D), jnp.float32),
    mesh=_mesh,
    scratch_types=[
        pltpu.VMEM((CS, K), jnp.int32),
        pltpu.VMEM((CS, K), jnp.int32),
        [pltpu.VMEM((K, D), jnp.float32)] * NB,
        [pltpu.SemaphoreType.DMA] * NB,
        [pltpu.SemaphoreType.DMA] * NB,
        pltpu.VMEM_SHARED((N, D), jnp.float32),
    ],
)
def _sc_aggregate(h_hbm, src_hbm, dst_hbm, zeros_hbm, out_hbm,
                  sidx, didx, rows, gsem, ssem, acc):
    cid = lax.axis_index("c")
    sid = lax.axis_index("s")
    w = cid * NS + sid

    def gather(c, b):
        return pltpu.async_copy(h_hbm.at[sidx.at[c]], rows[b], gsem[b])

    def gather_wait(c, b):
        pltpu.make_async_copy(h_hbm.at[sidx.at[c]], rows[b], gsem[b]).wait()

    def scat(c, b):
        return pltpu.async_copy(rows[b], acc.at[didx.at[c]], ssem[b],
                                add=True)

    def scat_wait(c, b):
        pltpu.make_async_copy(rows[b], acc.at[didx.at[c]], ssem[b]).wait()

    @pl.when(sid < IO_T)
    def _():
        pltpu.sync_copy(zeros_hbm, rows[0])
        for j in range(RPT // ZR):
            pltpu.async_copy(rows[0], acc.at[pl.ds(sid * RPT + j * ZR, ZR)],
                             ssem[0])
        for j in range(RPT // ZR):
            pltpu.make_async_copy(
                rows[0], acc.at[pl.ds(sid * RPT + j * ZR, ZR)], ssem[0]).wait()

    plsc.subcore_barrier()

    def ring_round(j, carry):
        base = NB * j
        for b in range(NB):
            gather_wait(base + b, b)
            scat(base + b, b)
        for b in range(NB):
            scat_wait(base + b, b)
            gather(base + NB + b, b)
        return carry

    for s in range(SEG):
        pltpu.sync_copy(src_hbm.at[w, s], sidx)
        pltpu.sync_copy(dst_hbm.at[w, s], didx)
        for b in range(NB):
            gather(b, b)
        J = CS // NB - 1
        lax.fori_loop(0, J, ring_round, 0)
        baseT = NB * J
        R = CS - baseT - NB
        for b in range(NB):
            gather_wait(baseT + b, b)
            scat(baseT + b, b)
        for b in range(R):
            scat_wait(baseT + b, b)
            gather(baseT + NB + b, b)
        for b in range(R):
            gather_wait(baseT + NB + b, b)
            scat(baseT + NB + b, b)
        for b in range(R, NB):
            scat_wait(baseT + b, b)
        for b in range(R):
            scat_wait(baseT + NB + b, b)
    plsc.subcore_barrier()

    @pl.when(sid < IO_T)
    def _():
        nj = RPT // ZR
        for j in range(nj):
            b, sm = rows[j % 2], ssem[j % 2]
            if j >= 2:
                pltpu.make_async_copy(
                    b, out_hbm.at[cid, pl.ds(sid * RPT + (j - 2) * ZR, ZR)],
                    sm).wait()
            pltpu.sync_copy(acc.at[pl.ds(sid * RPT + j * ZR, ZR)], b)
            pltpu.async_copy(
                b, out_hbm.at[cid, pl.ds(sid * RPT + j * ZR, ZR)], sm)
        for j in (nj - 2, nj - 1):
            pltpu.make_async_copy(
                rows[j % 2], out_hbm.at[cid, pl.ds(sid * RPT + j * ZR, ZR)],
                ssem[j % 2]).wait()


@functools.partial(
    pl.kernel,
    out_type=jax.ShapeDtypeStruct((NC * N,), jnp.float32),
    mesh=_mesh,
    scratch_types=[
        pltpu.VMEM((CD, KD), jnp.int32),
        pltpu.VMEM((KD,), jnp.float32),
        pltpu.VMEM((RPT,), jnp.float32),
        pltpu.VMEM_SHARED((N,), jnp.float32),
    ],
)
def _sc_degree(dst_hbm, zeros_hbm, out_hbm, didx, ones_v, dbuf, acc):
    cid = lax.axis_index("c")
    sid = lax.axis_index("s")
    w = cid * NS + sid

    @pl.when(sid < IO_T)
    def _():
        pltpu.sync_copy(zeros_hbm, dbuf)
        pltpu.sync_copy(dbuf, acc.at[pl.ds(sid * RPT, RPT)])

    pltpu.sync_copy(dst_hbm.at[w], didx)
    for i in range(KD // 16):
        ones_v[pl.ds(i * 16, 16)] = jnp.ones((16,), jnp.float32)
    plsc.subcore_barrier()

    def chunk(c, carry):
        pltpu.sync_copy(ones_v, acc.at[didx.at[c]], add=True)
        return carry

    lax.fori_loop(0, CD, chunk, 0)
    plsc.subcore_barrier()

    @pl.when(sid < IO_T)
    def _():
        pltpu.sync_copy(acc.at[pl.ds(sid * RPT, RPT)], dbuf)
        pltpu.sync_copy(dbuf, out_hbm.at[pl.ds(cid * N + sid * RPT, RPT)])


_BM = 1000


def _dinv_block(dref):
    d = dref[...]
    deg = d[:, 0:1] + d[:, 1:2] + 1.0
    return lax.rsqrt(jnp.maximum(deg, 1.0))


def _prep_body(xref, wref, dref, out):
    out[...] = jnp.dot(xref[...], wref[...],
                       preferred_element_type=jnp.float32) * _dinv_block(dref)


def _layer_body(aref, href, wref, bref, dref, out):
    dinv = _dinv_block(dref)
    p = aref[0] + aref[1] + href[...]
    t = jnp.maximum(p * dinv + bref[...], 0.0)
    out[...] = jnp.dot(t, wref[...],
                       preferred_element_type=jnp.float32) * dinv


def _final_body(aref, href, bref, dref, out):
    dinv = _dinv_block(dref)
    out[...] = (aref[0] + aref[1] + href[...]) * dinv + bref[...]


_row_spec = pl.BlockSpec((_BM, D), lambda i: (i, 0))
_agg_spec = pl.BlockSpec((NC, _BM, D), lambda i: (0, i, 0))
_w_spec = pl.BlockSpec((D, D), lambda i: (0, 0))
_b_spec = pl.BlockSpec((1, D), lambda i: (0, 0))
_deg_spec = pl.BlockSpec((_BM, NC), lambda i: (i, 0))
_out_f32 = jax.ShapeDtypeStruct((N, D), jnp.float32)

_prep = pl.pallas_call(
    _prep_body, grid=(N // _BM,),
    in_specs=[_row_spec, _w_spec, _deg_spec],
    out_specs=_row_spec, out_shape=_out_f32)

_layer = pl.pallas_call(
    _layer_body, grid=(N // _BM,),
    in_specs=[_agg_spec, _row_spec, _w_spec, _b_spec, _deg_spec],
    out_specs=_row_spec, out_shape=_out_f32)

_final = pl.pallas_call(
    _final_body, grid=(N // _BM,),
    in_specs=[_agg_spec, _row_spec, _b_spec, _deg_spec],
    out_specs=_row_spec, out_shape=_out_f32)


def kernel(x, edge_index, W1, b1, W2, b2, W3, b3):
    src = edge_index[0].reshape(NW, SEG, CS, K)
    dst = edge_index[1].reshape(NW, SEG, CS, K)
    dst_flat = edge_index[1].reshape(NW, CD, KD)
    zeros_rows = jnp.zeros((ZR, D), jnp.float32)
    zeros_deg = jnp.zeros((RPT,), jnp.float32)
    b1r = b1.reshape(1, D)
    b2r = b2.reshape(1, D)
    b3r = b3.reshape(1, D)

    degp = _sc_degree(dst_flat, zeros_deg).reshape(NC, N).T
    h1 = _prep(x, W1, degp)
    a1 = _sc_aggregate(h1, src, dst, zeros_rows)
    h2 = _layer(a1, h1, W2, b1r, degp)
    a2 = _sc_aggregate(h2, src, dst, zeros_rows)
    h3 = _layer(a2, h2, W3, b2r, degp)
    a3 = _sc_aggregate(h3, src, dst, zeros_rows)
    return _final(a3, h3, b3r, degp)
